# baseline (speedup 1.0000x reference)
import functools

import jax
import jax.numpy as jnp
from jax import lax
from jax.experimental import pallas as pl
from jax.experimental.pallas import tpu as pltpu
from jax.experimental.pallas import tpu_sc as plsc

_C = 256
_M = 128
_HW = 512 * 512
_K = 64
_D = _HW // _K
_SH = 6
_NW = 32
_ROWS_OUT = _M * _K
_ROWS_PER_TILE = _ROWS_OUT // _NW
_CHUNK = 8
_NCHUNK = _ROWS_PER_TILE // _CHUNK
_NPAIR = _NCHUNK // 2
_L = 16


def _sc_body(table_hbm, cm_hbm, out_hbm, cm_v, idx_v, b0, b1,
             gsem0, gsem1, ssem0, ssem1):
    wid = lax.axis_index("s") * 2 + lax.axis_index("c")
    base = wid * _ROWS_PER_TILE

    iota = lax.broadcasted_iota(jnp.int32, (_L,), 0)
    for v in range(_ROWS_PER_TILE // _L):
        r16 = base + v * _L + iota
        idx_v[pl.ds(v * _L, _L)] = lax.shift_right_logical(r16, _SH)
    pltpu.async_copy(cm_hbm.at[idx_v], cm_v, gsem0).wait()
    for v in range(_ROWS_PER_TILE // _L):
        r16 = base + v * _L + iota
        off = jnp.bitwise_and(r16, _K - 1)
        idx_v[pl.ds(v * _L, _L)] = cm_v[pl.ds(v * _L, _L)] * _K + off

    def g_start(c, buf, sem):
        pltpu.async_copy(table_hbm.at[idx_v.at[pl.ds(c * _CHUNK, _CHUNK)]],
                         buf, sem)

    def g_wait(buf, sem):
        pltpu.make_async_copy(table_hbm.at[idx_v.at[pl.ds(0, _CHUNK)]],
                              buf, sem).wait()

    def s_start(c, buf, sem):
        pltpu.async_copy(buf, out_hbm.at[pl.ds(base + c * _CHUNK, _CHUNK)],
                         sem)

    def s_wait(buf, sem):
        pltpu.make_async_copy(buf, out_hbm.at[pl.ds(base, _CHUNK)], sem).wait()

    g_start(0, b0, gsem0)

    def pair_body(i, carry):
        c0 = 2 * i

        @pl.when(i > 0)
        def _():
            s_wait(b1, ssem1)

        g_start(c0 + 1, b1, gsem1)
        g_wait(b0, gsem0)
        s_start(c0, b0, ssem0)

        @pl.when(i < _NPAIR - 1)
        def _():
            s_wait(b0, ssem0)
            g_start(c0 + 2, b0, gsem0)

        g_wait(b1, gsem1)
        s_start(c0 + 1, b1, ssem1)
        return carry

    lax.fori_loop(0, _NPAIR, pair_body, 0)
    s_wait(b0, ssem0)
    s_wait(b1, ssem1)


@jax.jit
def _sc_gather(table, channel_map):
    mesh = plsc.VectorSubcoreMesh(core_axis_name="c", subcore_axis_name="s")
    return pl.kernel(
        _sc_body,
        mesh=mesh,
        out_type=jax.ShapeDtypeStruct((_ROWS_OUT, _D), jnp.float32),
        scratch_types=[
            pltpu.VMEM((_ROWS_PER_TILE,), jnp.int32),
            pltpu.VMEM((_ROWS_PER_TILE,), jnp.int32),
            pltpu.VMEM((_CHUNK, _D), jnp.float32),
            pltpu.VMEM((_CHUNK, _D), jnp.float32),
            pltpu.SemaphoreType.DMA,
            pltpu.SemaphoreType.DMA,
            pltpu.SemaphoreType.DMA,
            pltpu.SemaphoreType.DMA,
        ],
    )(table, channel_map)


def kernel(model, channel_map):
    c, h, w = model.shape
    table = model.reshape(c * _K, (h * w) // _K)
    out = _sc_gather(table, channel_map.astype(jnp.int32))
    return out.reshape(channel_map.shape[0], h, w)


# speedup vs baseline: 4.0252x; 4.0252x over previous
import functools

import jax
import jax.numpy as jnp
from jax import lax
from jax.experimental import pallas as pl
from jax.experimental.pallas import tpu as pltpu
from jax.experimental.pallas import tpu_sc as plsc

_C = 256
_M = 128
_HW = 512 * 512
_K = 64
_SH = 6
_NW = 32
_ROWS_OUT = _M * _K
_ROWS_PER_TILE = _ROWS_OUT // _NW
_CHUNK = 8
_NCHUNK = _ROWS_PER_TILE // _CHUNK
_NPAIR = _NCHUNK // 2
_L = 16


def _sc_body(table_hbm, cm_hbm, out_hbm, cm_v, idx_v, b0, b1,
             gsem0, gsem1, ssem0, ssem1):
    wid = lax.axis_index("s") * 2 + lax.axis_index("c")
    base = wid * _ROWS_PER_TILE

    iota = lax.broadcasted_iota(jnp.int32, (_L,), 0)
    for v in range(_ROWS_PER_TILE // _L):
        r16 = base + v * _L + iota
        idx_v[pl.ds(v * _L, _L)] = lax.shift_right_logical(r16, _SH)
    pltpu.async_copy(cm_hbm.at[idx_v.at[pl.ds(0, 128)]],
                     cm_v.at[pl.ds(0, 128)], gsem0)
    pltpu.async_copy(cm_hbm.at[idx_v.at[pl.ds(128, 128)]],
                     cm_v.at[pl.ds(128, 128)], gsem1)
    pltpu.make_async_copy(cm_hbm.at[idx_v.at[pl.ds(0, 128)]],
                          cm_v.at[pl.ds(0, 128)], gsem0).wait()
    pltpu.make_async_copy(cm_hbm.at[idx_v.at[pl.ds(128, 128)]],
                          cm_v.at[pl.ds(128, 128)], gsem1).wait()
    for v in range(_ROWS_PER_TILE // _L):
        r16 = base + v * _L + iota
        off = jnp.bitwise_and(r16, _K - 1)
        idx_v[pl.ds(v * _L, _L)] = cm_v[pl.ds(v * _L, _L)] * _K + off

    def g_start(c, buf, sem):
        pltpu.async_copy(table_hbm.at[idx_v.at[pl.ds(c * _CHUNK, _CHUNK)]],
                         buf, sem)

    def g_wait(buf, sem):
        pltpu.make_async_copy(table_hbm.at[idx_v.at[pl.ds(0, _CHUNK)]],
                              buf, sem).wait()

    def s_start(c, buf, sem):
        pltpu.async_copy(buf, out_hbm.at[pl.ds(base + c * _CHUNK, _CHUNK)],
                         sem)

    def s_wait(buf, sem):
        pltpu.make_async_copy(buf, out_hbm.at[pl.ds(base, _CHUNK)], sem).wait()

    g_start(0, b0, gsem0)

    def pair_body(i, carry):
        c0 = 2 * i

        @pl.when(i > 0)
        def _():
            s_wait(b1, ssem1)

        g_start(c0 + 1, b1, gsem1)
        g_wait(b0, gsem0)
        s_start(c0, b0, ssem0)

        @pl.when(i < _NPAIR - 1)
        def _():
            s_wait(b0, ssem0)
            g_start(c0 + 2, b0, gsem0)

        g_wait(b1, gsem1)
        s_start(c0 + 1, b1, ssem1)
        return carry

    lax.fori_loop(0, _NPAIR, pair_body, 0)
    s_wait(b0, ssem0)
    s_wait(b1, ssem1)


@jax.jit
def _sc_gather(table, channel_map):
    mesh = plsc.VectorSubcoreMesh(core_axis_name="c", subcore_axis_name="s")
    return pl.kernel(
        _sc_body,
        mesh=mesh,
        out_type=jax.ShapeDtypeStruct((_ROWS_OUT, 8, 512), jnp.float32),
        scratch_types=[
            pltpu.VMEM((_ROWS_PER_TILE,), jnp.int32),
            pltpu.VMEM((_ROWS_PER_TILE,), jnp.int32),
            pltpu.VMEM((_CHUNK, 8, 512), jnp.float32),
            pltpu.VMEM((_CHUNK, 8, 512), jnp.float32),
            pltpu.SemaphoreType.DMA,
            pltpu.SemaphoreType.DMA,
            pltpu.SemaphoreType.DMA,
            pltpu.SemaphoreType.DMA,
        ],
    )(table, channel_map)


def kernel(model, channel_map):
    c, h, w = model.shape
    table = model.reshape(c * _K, 8, w)
    out = _sc_gather(table, channel_map.astype(jnp.int32))
    return out.reshape(channel_map.shape[0], h, w)
